# Initial kernel scaffold; baseline (speedup 1.0000x reference)
import jax
import jax.numpy as jnp
from jax.experimental import pallas as pl


def kernel(frame, flow):
    raise NotImplementedError("write your pallas kernel here")


# speedup vs baseline: 1.6845x; 1.6845x over previous
import functools

import jax
import jax.numpy as jnp
from jax import lax
from jax.experimental import pallas as pl
from jax.experimental.pallas import tpu as pltpu
from jax.experimental.pallas import tpu_sc as plsc

NC, NS, L = 2, 16, 16
B, C, H, W = 2, 96, 384, 384
NPIX = H * W
CPC = C // NC
PPT = NPIX // NS
RPT = H // NS
VPR = W // L
IDX_ROWS = PPT // 128
ZCHUNK = PPT // 4


def _splat_body(frame_ref, flow_ref, out_ref,
                idx0, idx1, idx2, idx3, wgt_s, vals_s, prod_s,
                fxr, fyr, zero_s, plane):
    core = lax.axis_index("c")
    sid = lax.axis_index("s")
    idx_refs = (idx0, idx1, idx2, idx3)
    pix0 = sid * PPT
    row0 = sid * RPT

    zvec = jnp.zeros((L,), jnp.float32)
    def _zb(i, _):
        zero_s[pl.ds(i * L, L)] = zvec
        return ()
    lax.fori_loop(0, ZCHUNK // L, _zb, ())

    iota = lax.iota(jnp.int32, L)

    for b in range(B):
        def _setup_row(hrow, _):
            h_img = row0 + hrow
            fy_base = (h_img).astype(jnp.float32)
            pltpu.sync_copy(
                flow_ref.at[b, 0, pl.ds(pix0 + hrow * W, W)], fxr)
            pltpu.sync_copy(
                flow_ref.at[b, 1, pl.ds(pix0 + hrow * W, W)], fyr)
            for v in range(VPR):
                fx = (v * L + iota).astype(jnp.float32) + fxr[pl.ds(v * L, L)]
                fy = fy_base + fyr[pl.ds(v * L, L)]
                tx = fx.astype(jnp.int32)
                ty = fy.astype(jnp.int32)
                x0 = tx - jnp.where(tx.astype(jnp.float32) > fx, 1, 0)
                y0 = ty - jnp.where(ty.astype(jnp.float32) > fy, 1, 0)
                dx = fx - x0.astype(jnp.float32)
                dy = fy - y0.astype(jnp.float32)
                omdx = 1.0 - dx
                omdy = 1.0 - dy
                off = hrow * W + v * L
                for k, (xi, yi, wk) in enumerate((
                        (x0, y0, omdx * omdy),
                        (x0 + 1, y0, dx * omdy),
                        (x0, y0 + 1, omdx * dy),
                        (x0 + 1, y0 + 1, dx * dy))):
                    valid = ((xi >= 0) & (xi < W) & (yi >= 0) & (yi < H))
                    xc = jnp.minimum(jnp.maximum(xi, 0), W - 1)
                    yc = jnp.minimum(jnp.maximum(yi, 0), H - 1)
                    idx_refs[k][pl.ds(off, L)] = yc * W + xc
                    wgt_s[k, pl.ds(off, L)] = jnp.where(valid, wk, 0.0)
            return ()
        lax.fori_loop(0, RPT, _setup_row, ())

        def _chan(ch, _):
            cg = core * CPC + ch
            for q in range(4):
                pltpu.sync_copy(zero_s,
                                plane.at[pl.ds(pix0 + q * ZCHUNK, ZCHUNK)])
            plsc.subcore_barrier()
            pltpu.sync_copy(frame_ref.at[b, cg, pl.ds(pix0, PPT)], vals_s)
            for k in range(4):
                def _prod_row(r2, _):
                    for u in range(8):
                        cc = r2 * 128 + u * L
                        prod_s[pl.ds(cc, L)] = (
                            vals_s[pl.ds(cc, L)] * wgt_s[k, pl.ds(cc, L)])
                    return ()
                lax.fori_loop(0, IDX_ROWS, _prod_row, ())
                pltpu.sync_copy(prod_s, plane.at[idx_refs[k]], add=True)
            plsc.subcore_barrier()
            pltpu.sync_copy(plane.at[pl.ds(pix0, PPT)],
                            out_ref.at[b, cg, pl.ds(pix0, PPT)])
            plsc.subcore_barrier()
            return ()
        lax.fori_loop(0, CPC, _chan, ())


@jax.jit
def _splat_sc(frame_flat, flow_flat):
    mesh = plsc.VectorSubcoreMesh(core_axis_name="c", subcore_axis_name="s",
                                  num_cores=NC, num_subcores=NS)
    return pl.kernel(
        _splat_body,
        out_type=jax.ShapeDtypeStruct((B, C, NPIX), jnp.float32),
        mesh=mesh,
        scratch_types=[
            pltpu.VMEM((PPT,), jnp.int32),
            pltpu.VMEM((PPT,), jnp.int32),
            pltpu.VMEM((PPT,), jnp.int32),
            pltpu.VMEM((PPT,), jnp.int32),
            pltpu.VMEM((4, PPT), jnp.float32),
            pltpu.VMEM((PPT,), jnp.float32),
            pltpu.VMEM((PPT,), jnp.float32),
            pltpu.VMEM((W,), jnp.float32),
            pltpu.VMEM((W,), jnp.float32),
            pltpu.VMEM((ZCHUNK,), jnp.float32),
            pltpu.VMEM_SHARED((NPIX,), jnp.float32),
        ],
    )(frame_flat, flow_flat)


def kernel(frame, flow):
    frame_flat = frame.reshape(B, C, NPIX)
    flow_flat = flow.reshape(B, 2, NPIX)
    out = _splat_sc(frame_flat, flow_flat)
    return out.reshape(B, C, H, W)


# speedup vs baseline: 2.3687x; 1.4062x over previous
import functools

import jax
import jax.numpy as jnp
from jax import lax
from jax.experimental import pallas as pl
from jax.experimental.pallas import tpu as pltpu
from jax.experimental.pallas import tpu_sc as plsc

NC, NS, L = 2, 16, 16
B, C, H, W = 2, 96, 384, 384
NPIX = H * W
CPC = C // NC
PPT = NPIX // NS
RPT = H // NS
VPR = W // L
IDX_ROWS = PPT // 128
ZCHUNK = PPT // 4


def _splat_body(frame_ref, flow_ref, out_ref,
                idx0, idx1, idx2, idx3, wgt_s, vals_s, prod0, prod1,
                fxr, fyr, zero_s, plane, sem0, sem1):
    core = lax.axis_index("c")
    sid = lax.axis_index("s")
    idx_refs = (idx0, idx1, idx2, idx3)
    pix0 = sid * PPT
    row0 = sid * RPT

    zvec = jnp.zeros((L,), jnp.float32)
    def _zb(i, _):
        zero_s[pl.ds(i * L, L)] = zvec
        return ()
    lax.fori_loop(0, ZCHUNK // L, _zb, ())

    iota = lax.iota(jnp.int32, L)

    for b in range(B):
        def _setup_row(hrow, _):
            h_img = row0 + hrow
            fy_base = (h_img).astype(jnp.float32)
            pltpu.sync_copy(
                flow_ref.at[b, 0, pl.ds(pix0 + hrow * W, W)], fxr)
            pltpu.sync_copy(
                flow_ref.at[b, 1, pl.ds(pix0 + hrow * W, W)], fyr)
            for v in range(VPR):
                fx = (v * L + iota).astype(jnp.float32) + fxr[pl.ds(v * L, L)]
                fy = fy_base + fyr[pl.ds(v * L, L)]
                tx = fx.astype(jnp.int32)
                ty = fy.astype(jnp.int32)
                x0 = tx - jnp.where(tx.astype(jnp.float32) > fx, 1, 0)
                y0 = ty - jnp.where(ty.astype(jnp.float32) > fy, 1, 0)
                dx = fx - x0.astype(jnp.float32)
                dy = fy - y0.astype(jnp.float32)
                omdx = 1.0 - dx
                omdy = 1.0 - dy
                off = hrow * W + v * L
                for k, (xi, yi, wk) in enumerate((
                        (x0, y0, omdx * omdy),
                        (x0 + 1, y0, dx * omdy),
                        (x0, y0 + 1, omdx * dy),
                        (x0 + 1, y0 + 1, dx * dy))):
                    valid = ((xi >= 0) & (xi < W) & (yi >= 0) & (yi < H))
                    xc = jnp.minimum(jnp.maximum(xi, 0), W - 1)
                    yc = jnp.minimum(jnp.maximum(yi, 0), H - 1)
                    idx_refs[k][pl.ds(off, L)] = yc * W + xc
                    wgt_s[k, pl.ds(off, L)] = jnp.where(valid, wk, 0.0)
            return ()
        lax.fori_loop(0, RPT, _setup_row, ())

        def _chan(ch, _):
            cg = core * CPC + ch
            for q in range(4):
                pltpu.sync_copy(zero_s,
                                plane.at[pl.ds(pix0 + q * ZCHUNK, ZCHUNK)])
            plsc.subcore_barrier()
            pltpu.sync_copy(frame_ref.at[b, cg, pl.ds(pix0, PPT)], vals_s)
            prods = (prod0, prod1)
            sems = (sem0, sem1)
            descs = {}
            for k in range(4):
                buf = prods[k % 2]
                if k >= 2:
                    descs[k - 2].wait()
                def _prod_row(r2, _):
                    for u in range(8):
                        cc = r2 * 128 + u * L
                        buf[pl.ds(cc, L)] = (
                            vals_s[pl.ds(cc, L)] * wgt_s[k, pl.ds(cc, L)])
                    return ()
                lax.fori_loop(0, IDX_ROWS, _prod_row, ())
                descs[k] = pltpu.async_copy(
                    buf, plane.at[idx_refs[k]], sems[k % 2], add=True)
            descs[2].wait()
            descs[3].wait()
            plsc.subcore_barrier()
            pltpu.sync_copy(plane.at[pl.ds(pix0, PPT)],
                            out_ref.at[b, cg, pl.ds(pix0, PPT)])
            plsc.subcore_barrier()
            return ()
        lax.fori_loop(0, CPC, _chan, ())


@jax.jit
def _splat_sc(frame_flat, flow_flat):
    mesh = plsc.VectorSubcoreMesh(core_axis_name="c", subcore_axis_name="s",
                                  num_cores=NC, num_subcores=NS)
    return pl.kernel(
        _splat_body,
        out_type=jax.ShapeDtypeStruct((B, C, NPIX), jnp.float32),
        mesh=mesh,
        scratch_types=[
            pltpu.VMEM((PPT,), jnp.int32),
            pltpu.VMEM((PPT,), jnp.int32),
            pltpu.VMEM((PPT,), jnp.int32),
            pltpu.VMEM((PPT,), jnp.int32),
            pltpu.VMEM((4, PPT), jnp.float32),
            pltpu.VMEM((PPT,), jnp.float32),
            pltpu.VMEM((PPT,), jnp.float32),
            pltpu.VMEM((PPT,), jnp.float32),
            pltpu.VMEM((W,), jnp.float32),
            pltpu.VMEM((W,), jnp.float32),
            pltpu.VMEM((ZCHUNK,), jnp.float32),
            pltpu.VMEM_SHARED((NPIX,), jnp.float32),
            pltpu.SemaphoreType.DMA,
            pltpu.SemaphoreType.DMA,
        ],
    )(frame_flat, flow_flat)


def kernel(frame, flow):
    frame_flat = frame.reshape(B, C, NPIX)
    flow_flat = flow.reshape(B, 2, NPIX)
    out = _splat_sc(frame_flat, flow_flat)
    return out.reshape(B, C, H, W)


# speedup vs baseline: 2.3867x; 1.0076x over previous
import functools

import jax
import jax.numpy as jnp
from jax import lax
from jax.experimental import pallas as pl
from jax.experimental.pallas import tpu as pltpu
from jax.experimental.pallas import tpu_sc as plsc

NC, NS, L = 2, 16, 16
B, C, H, W = 2, 96, 384, 384
NPIX = H * W
CPC = C // NC
PPT = NPIX // NS
RPT = H // NS
VPR = W // L
IDX_ROWS = PPT // 128
ZCHUNK = PPT


def _splat_body(frame_ref, flow_ref, out_ref,
                idx0, idx1, idx2, idx3, wgt_s, vals_s, prod0, prod1,
                fxr, fyr, zero_s, plane, sem0, sem1):
    core = lax.axis_index("c")
    sid = lax.axis_index("s")
    idx_refs = (idx0, idx1, idx2, idx3)
    pix0 = sid * PPT
    row0 = sid * RPT

    zvec = jnp.zeros((L,), jnp.float32)
    def _zb(i, _):
        zero_s[pl.ds(i * L, L)] = zvec
        return ()
    lax.fori_loop(0, ZCHUNK // L, _zb, ())

    iota = lax.iota(jnp.int32, L)

    for b in range(B):
        def _setup_row(hrow, _):
            h_img = row0 + hrow
            fy_base = (h_img).astype(jnp.float32)
            pltpu.sync_copy(
                flow_ref.at[b, 0, pl.ds(pix0 + hrow * W, W)], fxr)
            pltpu.sync_copy(
                flow_ref.at[b, 1, pl.ds(pix0 + hrow * W, W)], fyr)
            for v in range(VPR):
                fx = (v * L + iota).astype(jnp.float32) + fxr[pl.ds(v * L, L)]
                fy = fy_base + fyr[pl.ds(v * L, L)]
                tx = fx.astype(jnp.int32)
                ty = fy.astype(jnp.int32)
                x0 = tx - jnp.where(tx.astype(jnp.float32) > fx, 1, 0)
                y0 = ty - jnp.where(ty.astype(jnp.float32) > fy, 1, 0)
                dx = fx - x0.astype(jnp.float32)
                dy = fy - y0.astype(jnp.float32)
                omdx = 1.0 - dx
                omdy = 1.0 - dy
                off = hrow * W + v * L
                for k, (xi, yi, wk) in enumerate((
                        (x0, y0, omdx * omdy),
                        (x0 + 1, y0, dx * omdy),
                        (x0, y0 + 1, omdx * dy),
                        (x0 + 1, y0 + 1, dx * dy))):
                    valid = ((xi >= 0) & (xi < W) & (yi >= 0) & (yi < H))
                    xc = jnp.minimum(jnp.maximum(xi, 0), W - 1)
                    yc = jnp.minimum(jnp.maximum(yi, 0), H - 1)
                    idx_refs[k][pl.ds(off, L)] = yc * W + xc
                    wgt_s[k, pl.ds(off, L)] = jnp.where(valid, wk, 0.0)
            return ()
        lax.fori_loop(0, RPT, _setup_row, ())

        def _chan(ch, _):
            cg = core * CPC + ch
            pltpu.sync_copy(zero_s, plane.at[pl.ds(pix0, PPT)])
            plsc.subcore_barrier()
            pltpu.sync_copy(frame_ref.at[b, cg, pl.ds(pix0, PPT)], vals_s)
            prods = (prod0, prod1)
            sems = (sem0, sem1)
            descs = {}
            for k in range(4):
                buf = prods[k % 2]
                if k >= 2:
                    descs[k - 2].wait()
                def _prod_row(r2, _):
                    for u in range(8):
                        cc = r2 * 128 + u * L
                        buf[pl.ds(cc, L)] = (
                            vals_s[pl.ds(cc, L)] * wgt_s[k, pl.ds(cc, L)])
                    return ()
                lax.fori_loop(0, IDX_ROWS, _prod_row, ())
                descs[k] = pltpu.async_copy(
                    buf, plane.at[idx_refs[k]], sems[k % 2], add=True)
            descs[2].wait()
            descs[3].wait()
            plsc.subcore_barrier()
            pltpu.sync_copy(plane.at[pl.ds(pix0, PPT)],
                            out_ref.at[b, cg, pl.ds(pix0, PPT)])
            plsc.subcore_barrier()
            return ()
        lax.fori_loop(0, CPC, _chan, ())


@jax.jit
def _splat_sc(frame_flat, flow_flat):
    mesh = plsc.VectorSubcoreMesh(core_axis_name="c", subcore_axis_name="s",
                                  num_cores=NC, num_subcores=NS)
    return pl.kernel(
        _splat_body,
        out_type=jax.ShapeDtypeStruct((B, C, NPIX), jnp.float32),
        mesh=mesh,
        scratch_types=[
            pltpu.VMEM((PPT,), jnp.int32),
            pltpu.VMEM((PPT,), jnp.int32),
            pltpu.VMEM((PPT,), jnp.int32),
            pltpu.VMEM((PPT,), jnp.int32),
            pltpu.VMEM((4, PPT), jnp.float32),
            pltpu.VMEM((PPT,), jnp.float32),
            pltpu.VMEM((PPT,), jnp.float32),
            pltpu.VMEM((PPT,), jnp.float32),
            pltpu.VMEM((W,), jnp.float32),
            pltpu.VMEM((W,), jnp.float32),
            pltpu.VMEM((ZCHUNK,), jnp.float32),
            pltpu.VMEM_SHARED((NPIX,), jnp.float32),
            pltpu.SemaphoreType.DMA,
            pltpu.SemaphoreType.DMA,
        ],
    )(frame_flat, flow_flat)


def kernel(frame, flow):
    frame_flat = frame.reshape(B, C, NPIX)
    flow_flat = flow.reshape(B, 2, NPIX)
    out = _splat_sc(frame_flat, flow_flat)
    return out.reshape(B, C, H, W)


# speedup vs baseline: 2.3870x; 1.0001x over previous
import functools

import jax
import jax.numpy as jnp
from jax import lax
from jax.experimental import pallas as pl
from jax.experimental.pallas import tpu as pltpu
from jax.experimental.pallas import tpu_sc as plsc

NC, NS, L = 2, 16, 16
B, C, H, W = 2, 96, 384, 384
NPIX = H * W
CPC = C // NC
PPT = NPIX // NS
RPT = H // NS
VPR = W // L
IDX_ROWS = PPT // 128
ZCHUNK = PPT


def _splat_body(frame_ref, flow_ref, out_ref,
                idx0, idx1, idx2, idx3, wgt_s, vals_s,
                prod0, prod1, prod2,
                fxr, fyr, zero_s, plane, sem0, sem1, sem2):
    core = lax.axis_index("c")
    sid = lax.axis_index("s")
    idx_refs = (idx0, idx1, idx2, idx3)
    pix0 = sid * PPT
    row0 = sid * RPT

    zvec = jnp.zeros((L,), jnp.float32)
    def _zb(i, _):
        zero_s[pl.ds(i * L, L)] = zvec
        return ()
    lax.fori_loop(0, ZCHUNK // L, _zb, ())

    iota = lax.iota(jnp.int32, L)

    for b in range(B):
        def _setup_row(hrow, _):
            h_img = row0 + hrow
            fy_base = (h_img).astype(jnp.float32)
            pltpu.sync_copy(
                flow_ref.at[b, 0, pl.ds(pix0 + hrow * W, W)], fxr)
            pltpu.sync_copy(
                flow_ref.at[b, 1, pl.ds(pix0 + hrow * W, W)], fyr)
            for v in range(VPR):
                fx = (v * L + iota).astype(jnp.float32) + fxr[pl.ds(v * L, L)]
                fy = fy_base + fyr[pl.ds(v * L, L)]
                tx = fx.astype(jnp.int32)
                ty = fy.astype(jnp.int32)
                x0 = tx - jnp.where(tx.astype(jnp.float32) > fx, 1, 0)
                y0 = ty - jnp.where(ty.astype(jnp.float32) > fy, 1, 0)
                dx = fx - x0.astype(jnp.float32)
                dy = fy - y0.astype(jnp.float32)
                omdx = 1.0 - dx
                omdy = 1.0 - dy
                off = hrow * W + v * L
                for k, (xi, yi, wk) in enumerate((
                        (x0, y0, omdx * omdy),
                        (x0 + 1, y0, dx * omdy),
                        (x0, y0 + 1, omdx * dy),
                        (x0 + 1, y0 + 1, dx * dy))):
                    valid = ((xi >= 0) & (xi < W) & (yi >= 0) & (yi < H))
                    xc = jnp.minimum(jnp.maximum(xi, 0), W - 1)
                    yc = jnp.minimum(jnp.maximum(yi, 0), H - 1)
                    idx_refs[k][pl.ds(off, L)] = yc * W + xc
                    wgt_s[k, pl.ds(off, L)] = jnp.where(valid, wk, 0.0)
            return ()
        lax.fori_loop(0, RPT, _setup_row, ())

        def _chan(ch, _):
            cg = core * CPC + ch
            pltpu.sync_copy(zero_s, plane.at[pl.ds(pix0, PPT)])
            plsc.subcore_barrier()
            pltpu.sync_copy(frame_ref.at[b, cg, pl.ds(pix0, PPT)], vals_s)
            prods = (prod0, prod1, prod2)
            sems = (sem0, sem1, sem2)
            descs = {}
            for k in range(4):
                buf = prods[k % 3]
                if k >= 3:
                    descs[k - 3].wait()
                def _prod_row(r2, _):
                    for u in range(8):
                        cc = r2 * 128 + u * L
                        buf[pl.ds(cc, L)] = (
                            vals_s[pl.ds(cc, L)] * wgt_s[k, pl.ds(cc, L)])
                    return ()
                lax.fori_loop(0, IDX_ROWS, _prod_row, ())
                descs[k] = pltpu.async_copy(
                    buf, plane.at[idx_refs[k]], sems[k % 3], add=True)
            descs[1].wait()
            descs[2].wait()
            descs[3].wait()
            plsc.subcore_barrier()
            pltpu.sync_copy(plane.at[pl.ds(pix0, PPT)],
                            out_ref.at[b, cg, pl.ds(pix0, PPT)])
            plsc.subcore_barrier()
            return ()
        lax.fori_loop(0, CPC, _chan, ())


@jax.jit
def _splat_sc(frame_flat, flow_flat):
    mesh = plsc.VectorSubcoreMesh(core_axis_name="c", subcore_axis_name="s",
                                  num_cores=NC, num_subcores=NS)
    return pl.kernel(
        _splat_body,
        out_type=jax.ShapeDtypeStruct((B, C, NPIX), jnp.float32),
        mesh=mesh,
        scratch_types=[
            pltpu.VMEM((PPT,), jnp.int32),
            pltpu.VMEM((PPT,), jnp.int32),
            pltpu.VMEM((PPT,), jnp.int32),
            pltpu.VMEM((PPT,), jnp.int32),
            pltpu.VMEM((4, PPT), jnp.float32),
            pltpu.VMEM((PPT,), jnp.float32),
            pltpu.VMEM((PPT,), jnp.float32),
            pltpu.VMEM((PPT,), jnp.float32),
            pltpu.VMEM((PPT,), jnp.float32),
            pltpu.VMEM((W,), jnp.float32),
            pltpu.VMEM((W,), jnp.float32),
            pltpu.VMEM((ZCHUNK,), jnp.float32),
            pltpu.VMEM_SHARED((NPIX,), jnp.float32),
            pltpu.SemaphoreType.DMA,
            pltpu.SemaphoreType.DMA,
            pltpu.SemaphoreType.DMA,
        ],
    )(frame_flat, flow_flat)


def kernel(frame, flow):
    frame_flat = frame.reshape(B, C, NPIX)
    flow_flat = flow.reshape(B, 2, NPIX)
    out = _splat_sc(frame_flat, flow_flat)
    return out.reshape(B, C, H, W)


# speedup vs baseline: 2.3898x; 1.0012x over previous
import functools

import jax
import jax.numpy as jnp
from jax import lax
from jax.experimental import pallas as pl
from jax.experimental.pallas import tpu as pltpu
from jax.experimental.pallas import tpu_sc as plsc

NC, NS, L = 2, 16, 16
B, C, H, W = 2, 96, 384, 384
NPIX = H * W
CPC = C // NC
PPT = NPIX // NS
RPT = H // NS
VPR = W // L
IDX_ROWS = PPT // 128
ZCHUNK = PPT


def _splat_body(frame_ref, flow_ref, out_ref,
                idx0, idx1, idx2, idx3, wgt_s, vals_s,
                prod0, prod1, prod2,
                fxr, fyr, zero_s, plane, sem0, sem1, sem2):
    core = lax.axis_index("c")
    sid = lax.axis_index("s")
    idx_refs = (idx0, idx1, idx2, idx3)
    pix0 = sid * PPT
    row0 = sid * RPT

    zvec = jnp.zeros((L,), jnp.float32)
    def _zb(i, _):
        zero_s[pl.ds(i * L, L)] = zvec
        return ()
    lax.fori_loop(0, ZCHUNK // L, _zb, ())

    iota = lax.iota(jnp.int32, L)

    for b in range(B):
        def _setup_row(hrow, _):
            h_img = row0 + hrow
            fy_base = (h_img).astype(jnp.float32)
            pltpu.sync_copy(
                flow_ref.at[b, 0, pl.ds(pix0 + hrow * W, W)], fxr)
            pltpu.sync_copy(
                flow_ref.at[b, 1, pl.ds(pix0 + hrow * W, W)], fyr)
            for v in range(VPR):
                fx = (v * L + iota).astype(jnp.float32) + fxr[pl.ds(v * L, L)]
                fy = fy_base + fyr[pl.ds(v * L, L)]
                tx = fx.astype(jnp.int32)
                ty = fy.astype(jnp.int32)
                x0 = tx - jnp.where(tx.astype(jnp.float32) > fx, 1, 0)
                y0 = ty - jnp.where(ty.astype(jnp.float32) > fy, 1, 0)
                dx = fx - x0.astype(jnp.float32)
                dy = fy - y0.astype(jnp.float32)
                omdx = 1.0 - dx
                omdy = 1.0 - dy
                off = hrow * W + v * L
                for k, (xi, yi, wk) in enumerate((
                        (x0, y0, omdx * omdy),
                        (x0 + 1, y0, dx * omdy),
                        (x0, y0 + 1, omdx * dy),
                        (x0 + 1, y0 + 1, dx * dy))):
                    valid = ((xi >= 0) & (xi < W) & (yi >= 0) & (yi < H))
                    xc = jnp.minimum(jnp.maximum(xi, 0), W - 1)
                    yc = jnp.minimum(jnp.maximum(yi, 0), H - 1)
                    idx_refs[k][pl.ds(off, L)] = yc * W + xc
                    wgt_s[k, pl.ds(off, L)] = jnp.where(valid, wk, 0.0)
            return ()
        lax.fori_loop(0, RPT, _setup_row, ())

        pltpu.sync_copy(zero_s, plane.at[pl.ds(pix0, PPT)])
        plsc.subcore_barrier()

        def _chan(ch, _):
            cg = core * CPC + ch
            pltpu.sync_copy(frame_ref.at[b, cg, pl.ds(pix0, PPT)], vals_s)
            prods = (prod0, prod1, prod2)
            sems = (sem0, sem1, sem2)
            descs = {}
            for k in range(4):
                buf = prods[k % 3]
                if k >= 3:
                    descs[k - 3].wait()
                def _prod_row(r2, _):
                    for u in range(8):
                        cc = r2 * 128 + u * L
                        buf[pl.ds(cc, L)] = (
                            vals_s[pl.ds(cc, L)] * wgt_s[k, pl.ds(cc, L)])
                    return ()
                lax.fori_loop(0, IDX_ROWS, _prod_row, ())
                descs[k] = pltpu.async_copy(
                    buf, plane.at[idx_refs[k]], sems[k % 3], add=True)
            descs[1].wait()
            descs[2].wait()
            descs[3].wait()
            plsc.subcore_barrier()
            pltpu.sync_copy(plane.at[pl.ds(pix0, PPT)],
                            out_ref.at[b, cg, pl.ds(pix0, PPT)])
            pltpu.sync_copy(zero_s, plane.at[pl.ds(pix0, PPT)])
            plsc.subcore_barrier()
            return ()
        lax.fori_loop(0, CPC, _chan, ())


@jax.jit
def _splat_sc(frame_flat, flow_flat):
    mesh = plsc.VectorSubcoreMesh(core_axis_name="c", subcore_axis_name="s",
                                  num_cores=NC, num_subcores=NS)
    return pl.kernel(
        _splat_body,
        out_type=jax.ShapeDtypeStruct((B, C, NPIX), jnp.float32),
        mesh=mesh,
        scratch_types=[
            pltpu.VMEM((PPT,), jnp.int32),
            pltpu.VMEM((PPT,), jnp.int32),
            pltpu.VMEM((PPT,), jnp.int32),
            pltpu.VMEM((PPT,), jnp.int32),
            pltpu.VMEM((4, PPT), jnp.float32),
            pltpu.VMEM((PPT,), jnp.float32),
            pltpu.VMEM((PPT,), jnp.float32),
            pltpu.VMEM((PPT,), jnp.float32),
            pltpu.VMEM((PPT,), jnp.float32),
            pltpu.VMEM((W,), jnp.float32),
            pltpu.VMEM((W,), jnp.float32),
            pltpu.VMEM((ZCHUNK,), jnp.float32),
            pltpu.VMEM_SHARED((NPIX,), jnp.float32),
            pltpu.SemaphoreType.DMA,
            pltpu.SemaphoreType.DMA,
            pltpu.SemaphoreType.DMA,
        ],
    )(frame_flat, flow_flat)


def kernel(frame, flow):
    frame_flat = frame.reshape(B, C, NPIX)
    flow_flat = flow.reshape(B, 2, NPIX)
    out = _splat_sc(frame_flat, flow_flat)
    return out.reshape(B, C, H, W)
